# Initial kernel scaffold; baseline (speedup 1.0000x reference)
import jax
import jax.numpy as jnp
from jax.experimental import pallas as pl


def kernel(query, relpos, emb_weight):
    raise NotImplementedError("write your pallas kernel here")


# speedup vs baseline: 1128.8630x; 1128.8630x over previous
import jax
import jax.numpy as jnp
from jax.experimental import pallas as pl
from jax.experimental.pallas import tpu as pltpu

Q_BLK = 256
K_BLK = 512


def _body(q_ref, rp_ref, w_ref, out_ref, s_ref):
    ik = pl.program_id(1)

    @pl.when(ik == 0)
    def _():
        for h in range(12):
            s_ref[h] = jnp.dot(q_ref[h], w_ref[h],
                               preferred_element_type=jnp.float32)

    rp = rp_ref[...]
    acc = jnp.zeros((12, Q_BLK, K_BLK), jnp.float32)
    for n in range(1, 22):
        s_n = s_ref[:, :, n]
        acc = jnp.where(rp[None, :, :] == n, s_n[:, :, None], acc)
    out_ref[...] = acc


def kernel(query, relpos, emb_weight):
    B, H, Q, dh = query.shape
    K = relpos.shape[2]
    n_emb = emb_weight.shape[0]

    qv = query.reshape(H, Q, dh)
    rp = relpos.reshape(Q, K)
    wt = emb_weight.reshape(n_emb, H, dh).transpose(1, 2, 0)

    grid = (Q // Q_BLK, K // K_BLK)
    out = pl.pallas_call(
        _body,
        grid=grid,
        in_specs=[
            pl.BlockSpec((H, Q_BLK, dh), lambda iq, ik: (0, iq, 0)),
            pl.BlockSpec((Q_BLK, K_BLK), lambda iq, ik: (iq, ik)),
            pl.BlockSpec((H, dh, n_emb), lambda iq, ik: (0, 0, 0)),
        ],
        out_specs=pl.BlockSpec((H, Q_BLK, K_BLK), lambda iq, ik: (0, iq, ik)),
        out_shape=jax.ShapeDtypeStruct((H, Q, K), jnp.float32),
        scratch_shapes=[pltpu.VMEM((H, Q_BLK, n_emb), jnp.float32)],
    )(qv, rp, wt)
    return out.reshape(B, H, Q, K)


# speedup vs baseline: 1178.8216x; 1.0443x over previous
import functools
import jax
import jax.numpy as jnp
from jax import lax
from jax.experimental import pallas as pl
from jax.experimental.pallas import tpu as pltpu
from jax.experimental.pallas import tpu_sc as plsc

NH = 12
NPAD = 32
LL = 16


def _scores_body(q_ref, w_ref, out_ref):
    parts = []
    for h in range(NH):
        parts.append(jnp.dot(q_ref[h], w_ref[h],
                             preferred_element_type=jnp.float32))
    out_ref[...] = jnp.concatenate(parts, axis=-1)


def _compute_scores(qv, wt):
    Q = qv.shape[1]
    return pl.pallas_call(
        _scores_body,
        out_shape=jax.ShapeDtypeStruct((Q, NH * NPAD), jnp.float32),
    )(qv, wt)


def _make_sc_gather(Q, K):
    info = plsc.get_sparse_core_info()
    NC, NS = info.num_cores, info.num_subcores
    NW = NC * NS
    QW = Q // NW
    mesh = plsc.VectorSubcoreMesh(core_axis_name="c", subcore_axis_name="s")

    @functools.partial(
        pl.kernel,
        mesh=mesh,
        compiler_params=pltpu.CompilerParams(needs_layout_passes=False),
        out_type=jax.ShapeDtypeStruct((NH, Q, K), jnp.float32),
        scratch_types=[
            pltpu.VMEM((QW * NH * NPAD,), jnp.float32),
            pltpu.VMEM((K,), jnp.int32),
            pltpu.VMEM((NH, K), jnp.float32),
        ],
    )
    def sc_gather(scores_hbm, rp_hbm, out_hbm, tbl_v, idx_v, obuf_v):
        wid = lax.axis_index("s") * NC + lax.axis_index("c")
        base = wid * QW
        pltpu.sync_copy(scores_hbm.at[pl.ds(base * NH * NPAD, QW * NH * NPAD)],
                        tbl_v)

        def q_body(qi, carry):
            pltpu.sync_copy(rp_hbm.at[pl.ds((base + qi) * K, K)], idx_v)
            qoff = qi * (NH * NPAD)

            def c_body(c, carry2):
                iv = idx_v[pl.ds(c * LL, LL)]
                for h in range(NH):
                    vals = plsc.load_gather(tbl_v, [iv + (qoff + h * NPAD)])
                    obuf_v[h, pl.ds(c * LL, LL)] = vals
                return carry2

            lax.fori_loop(0, K // LL, c_body, 0)
            pltpu.sync_copy(obuf_v, out_hbm.at[:, base + qi])
            return carry

        lax.fori_loop(0, QW, q_body, 0)

    return sc_gather


def kernel(query, relpos, emb_weight):
    B, H, Q, dh = query.shape
    K = relpos.shape[2]
    n_emb = emb_weight.shape[0]

    wt = emb_weight.at[0].set(0.0).reshape(n_emb, H, dh).transpose(1, 2, 0)
    wt = jnp.pad(wt, ((0, 0), (0, 0), (0, NPAD - n_emb)))
    qv = query.reshape(H, Q, dh)

    scores = _compute_scores(qv, wt).reshape(Q * H * NPAD)
    rp = relpos.reshape(Q * K)
    out = _make_sc_gather(Q, K)(scores, rp)
    return out.reshape(B, H, Q, K)


# speedup vs baseline: 5552.3832x; 4.7101x over previous
import functools
import jax
import jax.numpy as jnp
from jax import lax
from jax.experimental import pallas as pl
from jax.experimental.pallas import tpu as pltpu
from jax.experimental.pallas import tpu_sc as plsc

NH = 12
NPAD = 32
LL = 16


def _scores_body(q_ref, w_ref, out_ref):
    parts = []
    for h in range(NH):
        parts.append(jnp.dot(q_ref[h], w_ref[h],
                             preferred_element_type=jnp.float32))
    out_ref[...] = jnp.concatenate(parts, axis=-1)


def _compute_scores(qv, wt):
    Q = qv.shape[1]
    return pl.pallas_call(
        _scores_body,
        out_shape=jax.ShapeDtypeStruct((Q, NH * NPAD), jnp.float32),
    )(qv, wt)


def _make_sc_gather(Q, K):
    info = plsc.get_sparse_core_info()
    NC, NS = info.num_cores, info.num_subcores
    NW = NC * NS
    QW = Q // NW
    mesh = plsc.VectorSubcoreMesh(core_axis_name="c", subcore_axis_name="s")

    @functools.partial(
        pl.kernel,
        mesh=mesh,
        compiler_params=pltpu.CompilerParams(needs_layout_passes=False),
        out_type=jax.ShapeDtypeStruct((NH, Q, K), jnp.float32),
        scratch_types=[
            pltpu.VMEM((QW, NH * NPAD), jnp.float32),
            pltpu.VMEM((2, K), jnp.int32),
            pltpu.VMEM((2, NH, K), jnp.float32),
            pltpu.SemaphoreType.DMA,
            pltpu.SemaphoreType.DMA,
            pltpu.SemaphoreType.DMA,
            pltpu.SemaphoreType.DMA,
        ],
    )
    def sc_gather(scores_hbm, rp_hbm, out_hbm, tbl_v, idx_v, obuf_v,
                  si0, si1, so0, so1):
        sin = (si0, si1)
        sout = (so0, so1)
        wid = lax.axis_index("s") * NC + lax.axis_index("c")
        base = wid * QW
        pltpu.sync_copy(scores_hbm.at[pl.ds(base, QW)], tbl_v)

        def idx_dma(qi, b):
            return pltpu.make_async_copy(
                rp_hbm.at[pl.ds((base + qi) * K, K)], idx_v.at[b], sin[b])

        def out_dma(qi, b):
            return pltpu.make_async_copy(
                obuf_v.at[b], out_hbm.at[:, base + qi], sout[b])

        idx_dma(0, 0).start()
        idx_dma(1, 1).start()

        def q_pair(p, carry):
            for b in range(2):
                qi = p * 2 + b
                idx_dma(qi, b).wait()

                @pl.when(qi >= 2)
                def _():
                    out_dma(qi - 2, b).wait()

                @plsc.parallel_loop(0, K // LL, unroll=2)
                def _(c):
                    iv = idx_v[b, pl.ds(c * LL, LL)]
                    for h in range(NH):
                        obuf_v[b, h, pl.ds(c * LL, LL)] = plsc.load_gather(
                            tbl_v.at[qi, pl.ds(h * NPAD, NPAD)], [iv])
                out_dma(qi, b).start()

                @pl.when(qi + 2 < QW)
                def _():
                    idx_dma(qi + 2, b).start()
            return carry

        lax.fori_loop(0, QW // 2, q_pair, 0)
        out_dma(QW - 2, 0).wait()
        out_dma(QW - 1, 1).wait()

    return sc_gather


def kernel(query, relpos, emb_weight):
    B, H, Q, dh = query.shape
    K = relpos.shape[2]
    n_emb = emb_weight.shape[0]

    wt = emb_weight.at[0].set(0.0).reshape(n_emb, H, dh).transpose(1, 2, 0)
    wt = jnp.pad(wt, ((0, 0), (0, 0), (0, NPAD - n_emb)))
    qv = query.reshape(H, Q, dh)

    scores = _compute_scores(qv, wt)
    rp = relpos.reshape(Q * K)
    out = _make_sc_gather(Q, K)(scores, rp)
    return out.reshape(B, H, Q, K)
